# baseline (speedup 1.0000x reference)
import functools

import jax
import jax.numpy as jnp
from jax import lax
from jax.experimental import pallas as pl
from jax.experimental.pallas import tpu as pltpu
from jax.experimental.pallas import tpu_sc as plsc

_VOCAB = 1000000
_B = 4096
_L = 200
_D = 64
_NW = 32
_BPW = _B // _NW
_NG = _L // 16

_GATHER_DNUMS = lax.GatherDimensionNumbers(
    offset_dims=(), collapsed_slice_dims=(0,), start_index_map=(0,)
)


def _vgather(x, idx):
    return lax.gather(
        x,
        idx[:, None],
        _GATHER_DNUMS,
        slice_sizes=(1,),
        mode=lax.GatherScatterMode.PROMISE_IN_BOUNDS,
    )


def _make_sc_call():
    mesh = plsc.VectorSubcoreMesh(core_axis_name="c", subcore_axis_name="s")

    @functools.partial(
        pl.kernel,
        out_type=jax.ShapeDtypeStruct((_B, _L, _D), jnp.float32),
        mesh=mesh,
        scratch_types=[
            pltpu.VMEM((_BPW, _L), jnp.int32),
            pltpu.VMEM((_L, _D // 4), jnp.int32),
            pltpu.VMEM((_L, _D // 4), jnp.int32),
            pltpu.VMEM((_L,), jnp.float32),
            pltpu.VMEM((_L,), jnp.float32),
            pltpu.VMEM((_L, _D), jnp.float32),
            pltpu.VMEM((_L, _D), jnp.float32),
            pltpu.SemaphoreType.DMA,
            pltpu.SemaphoreType.DMA,
            pltpu.SemaphoreType.DMA,
            pltpu.SemaphoreType.DMA,
        ],
        compiler_params=pltpu.CompilerParams(
            needs_layout_passes=False, use_tc_tiling_on_sc=False
        ),
    )
    def sc_kernel(
        idx_hbm, tab_hbm, scl_hbm, out_hbm,
        idx_v, rows0, rows1, scl0, scl1, outv0, outv1,
        gsem0, gsem1, osem0, osem1,
    ):
        rows_b = (rows0, rows1)
        scl_b = (scl0, scl1)
        outv_b = (outv0, outv1)
        gsem_b = (gsem0, gsem1)
        osem_b = (osem0, osem1)


        wid = lax.axis_index("s") * 2 + lax.axis_index("c")
        lane = lax.iota(jnp.int32, 16)
        shl = (3 - (lane & 3)) << 3
        lane24 = jnp.full((16,), 24, jnp.int32)
        word_sel_k = [(lane >> 2) + 4 * k for k in range(4)]
        splat_const = [jnp.full((16,), ri, jnp.int32) for ri in range(16)]
        gcols = [16 * j for j in range(_NG)] + [_L - 16]

        pltpu.sync_copy(
            idx_hbm.at[pl.ds(pl.multiple_of(wid * _BPW, 8), _BPW)], idx_v
        )

        def fire(c, b):
            for lo, n in ((0, 128), (128, _L - 128)):
                pltpu.async_copy(
                    tab_hbm.at[idx_v.at[c, pl.ds(lo, n)]],
                    rows_b[b].at[pl.ds(lo, n)],
                    gsem_b[b],
                )
                pltpu.async_copy(
                    scl_hbm.at[idx_v.at[c, pl.ds(lo, n)]],
                    scl_b[b].at[pl.ds(lo, n)],
                    gsem_b[b],
                )

        def drain_gathers(b):
            pltpu.make_async_copy(
                tab_hbm.at[pl.ds(0, _L)], rows_b[b], gsem_b[b]
            ).wait()
            pltpu.make_async_copy(
                scl_hbm.at[pl.ds(0, _L)], scl_b[b], gsem_b[b]
            ).wait()

        def compute(c, b):
            rows_v, scl_v, out_v = rows_b[b], scl_b[b], outv_b[b]

            def do_rows(col, ri_lo):
                s16 = scl_v[pl.ds(col, 16)]
                for ri in range(ri_lo, 16):
                    rr = col + ri
                    s = _vgather(s16, splat_const[ri])
                    w = rows_v[rr]
                    for k in range(4):
                        wk = _vgather(w, word_sel_k[k])
                        bts = lax.shift_right_arithmetic(
                            lax.shift_left(wk, shl), lane24
                        )
                        out_v[rr, pl.ds(16 * k, 16)] = (
                            bts.astype(jnp.float32) * s
                        )

            def group_body(g, _):
                do_rows(g * 16, 0)
                return 0

            lax.fori_loop(0, _NG, group_body, 0)
            do_rows(_L - 16, 8)

        def fire_out(c, b):
            pltpu.async_copy(
                outv_b[b], out_hbm.at[wid * _BPW + c], osem_b[b]
            )

        def drain_out(b):
            pltpu.make_async_copy(
                outv_b[b], out_hbm.at[0], osem_b[b]
            ).wait()

        fire(0, 0)

        def pipe_body(i, _):
            for b in range(2):
                c = 2 * i + b
                drain_gathers(b)

                @pl.when(c + 1 < _BPW)
                def _():
                    fire(c + 1, b ^ 1)

                @pl.when(c >= 2)
                def _():
                    drain_out(b)

                compute(c, b)
                fire_out(c, b)
            return 0

        lax.fori_loop(0, _BPW // 2, pipe_body, 0)
        drain_out(0)
        drain_out(1)

    return sc_kernel


_SC_CALL = _make_sc_call()


def kernel(indices, weight, scales):
    tab32 = lax.bitcast_convert_type(
        weight.reshape(_VOCAB, _D // 4, 4), jnp.int32
    )
    return _SC_CALL(indices, tab32, scales)


# speedup vs baseline: 1.2381x; 1.2381x over previous
import functools

import jax
import jax.numpy as jnp
from jax import lax
from jax.experimental import pallas as pl
from jax.experimental.pallas import tpu as pltpu
from jax.experimental.pallas import tpu_sc as plsc

_VOCAB = 1000000
_B = 4096
_L = 200
_D = 64
_NW = 32
_BPW = _B // _NW
_NG = _L // 16

_GATHER_DNUMS = lax.GatherDimensionNumbers(
    offset_dims=(), collapsed_slice_dims=(0,), start_index_map=(0,)
)


def _vgather(x, idx):
    return lax.gather(
        x,
        idx[:, None],
        _GATHER_DNUMS,
        slice_sizes=(1,),
        mode=lax.GatherScatterMode.PROMISE_IN_BOUNDS,
    )


def _make_sc_call():
    mesh = plsc.VectorSubcoreMesh(core_axis_name="c", subcore_axis_name="s")

    @functools.partial(
        pl.kernel,
        out_type=jax.ShapeDtypeStruct((_B, _L, _D), jnp.float32),
        mesh=mesh,
        scratch_types=[
            pltpu.VMEM((_BPW, _L), jnp.int32),
            pltpu.VMEM((_L, _D // 4), jnp.int32),
            pltpu.VMEM((_L, _D // 4), jnp.int32),
            pltpu.VMEM((_L,), jnp.float32),
            pltpu.VMEM((_L,), jnp.float32),
            pltpu.VMEM((_L, _D), jnp.float32),
            pltpu.VMEM((_L, _D), jnp.float32),
            pltpu.SemaphoreType.DMA,
            pltpu.SemaphoreType.DMA,
            pltpu.SemaphoreType.DMA,
            pltpu.SemaphoreType.DMA,
        ],
        compiler_params=pltpu.CompilerParams(
            needs_layout_passes=False, use_tc_tiling_on_sc=False
        ),
    )
    def sc_kernel(
        idx_hbm, tab_hbm, scl_hbm, out_hbm,
        idx_v, rows0, rows1, scl0, scl1, outv0, outv1,
        gsem0, gsem1, osem0, osem1,
    ):
        rows_b = (rows0, rows1)
        scl_b = (scl0, scl1)
        outv_b = (outv0, outv1)
        gsem_b = (gsem0, gsem1)
        osem_b = (osem0, osem1)


        wid = lax.axis_index("s") * 2 + lax.axis_index("c")
        lane = lax.iota(jnp.int32, 16)
        shl = (3 - (lane & 3)) << 3
        lane24 = jnp.full((16,), 24, jnp.int32)
        word_sel_k = [(lane >> 2) + 4 * k for k in range(4)]
        splat_const = [jnp.full((16,), ri, jnp.int32) for ri in range(16)]
        gcols = [16 * j for j in range(_NG)] + [_L - 16]

        pltpu.sync_copy(
            idx_hbm.at[pl.ds(pl.multiple_of(wid * _BPW, 8), _BPW)], idx_v
        )

        def fire(c, b):
            for lo, n in ((0, 128), (128, _L - 128)):
                pltpu.async_copy(
                    tab_hbm.at[idx_v.at[c, pl.ds(lo, n)]],
                    rows_b[b].at[pl.ds(lo, n)],
                    gsem_b[b],
                )
                pltpu.async_copy(
                    scl_hbm.at[idx_v.at[c, pl.ds(lo, n)]],
                    scl_b[b].at[pl.ds(lo, n)],
                    gsem_b[b],
                )

        def drain_gathers(b):
            pltpu.make_async_copy(
                tab_hbm.at[pl.ds(0, _L)], rows_b[b], gsem_b[b]
            ).wait()
            pltpu.make_async_copy(
                scl_hbm.at[pl.ds(0, _L)], scl_b[b], gsem_b[b]
            ).wait()

        def compute(c, b):
            rows_v, scl_v, out_v = rows_b[b], scl_b[b], outv_b[b]

            def do_rows(col, ri_lo):
                s16 = scl_v[pl.ds(col, 16)]
                for ri in range(ri_lo, 16):
                    rr = col + ri
                    s = _vgather(s16, splat_const[ri])
                    w = rows_v[rr]
                    for k in range(4):
                        wk = _vgather(w, word_sel_k[k])
                        bts = lax.shift_right_arithmetic(
                            lax.shift_left(wk, shl), lane24
                        )
                        out_v[rr, pl.ds(16 * k, 16)] = (
                            bts.astype(jnp.float32) * s
                        )

            def group_body(g, _):
                do_rows(g * 16, 0)
                return 0

            lax.fori_loop(0, _NG, group_body, 0)
            do_rows(_L - 16, 8)

        def fire_out(c, b):
            pltpu.async_copy(
                outv_b[b], out_hbm.at[wid * _BPW + c], osem_b[b]
            )

        def drain_out(b):
            pltpu.make_async_copy(
                outv_b[b], out_hbm.at[0], osem_b[b]
            ).wait()

        fire(0, 0)

        def pipe_body(i, _):
            for b in range(2):
                c = 2 * i + b
                drain_gathers(b)

                @pl.when(c + 1 < _BPW)
                def _():
                    fire(c + 1, b ^ 1)

                @pl.when(c >= 2)
                def _():
                    drain_out(b)

                compute(c, b)
                fire_out(c, b)
            return 0

        lax.fori_loop(0, _BPW // 2, pipe_body, 0)
        drain_out(0)
        drain_out(1)

    return sc_kernel


_SC_CALL = _make_sc_call()

def _make_repack_call():
    mesh = plsc.VectorSubcoreMesh(core_axis_name="c", subcore_axis_name="s")
    rows_per_worker = _VOCAB // _NW
    chunk = 1250

    @functools.partial(
        pl.kernel,
        out_type=jax.ShapeDtypeStruct((_VOCAB, _D // 4), jnp.int32),
        mesh=mesh,
        scratch_types=[
            pltpu.VMEM((chunk, _D), jnp.int8),
            pltpu.VMEM((chunk, _D // 4), jnp.int32),
            pltpu.SemaphoreType.DMA,
        ],
        compiler_params=pltpu.CompilerParams(
            needs_layout_passes=False, use_tc_tiling_on_sc=False
        ),
    )
    def repack(tab8_hbm, tab32_hbm, v8, v32, sem):
        wid = lax.axis_index("s") * 2 + lax.axis_index("c")
        base = wid * rows_per_worker

        def chunk_body(cc, _):
            lo = base + cc * chunk
            pltpu.sync_copy(tab8_hbm.at[pl.ds(lo, chunk)], v8)

            def row_body(r, _):
                v32[r] = plsc.bitcast(v8[r], jnp.int32)
                return 0

            lax.fori_loop(0, chunk, row_body, 0, unroll=4)
            pltpu.sync_copy(v32, tab32_hbm.at[pl.ds(lo, chunk)])
            return 0

        lax.fori_loop(0, rows_per_worker // chunk, chunk_body, 0)

    return repack


_REPACK = _make_repack_call()


def kernel(indices, weight, scales):
    tab32 = _REPACK(weight)
    return _SC_CALL(indices, tab32, scales)
